# baseline (speedup 1.0000x reference)
import functools

import jax
import jax.numpy as jnp
from jax import lax
from jax.experimental import pallas as pl
from jax.experimental.pallas import tpu as pltpu
from jax.experimental.pallas import tpu_sc as plsc

PRE_SEQ_LEN = 128
HIDDEN = 768
EMB_DIM = 24 * HIDDEN
BATCH = 32
ROWS = BATCH * PRE_SEQ_LEN

CACHED_R = 64


def _sc_gather(table1, tpad1, pref):
    info = plsc.get_sparse_core_info()
    nc, ns = info.num_cores, info.num_subcores
    nw = nc * ns
    rows_per_w = ROWS // nw
    lr = CACHED_R // ns
    mesh = plsc.VectorSubcoreMesh(core_axis_name="c", subcore_axis_name="s")

    @functools.partial(
        pl.kernel,
        out_type=jax.ShapeDtypeStruct((ROWS * EMB_DIM,), jnp.float32),
        mesh=mesh,
        scratch_types=[
            pltpu.VMEM_SHARED((CACHED_R * EMB_DIM,), jnp.float32),
            pltpu.VMEM((rows_per_w, 16), jnp.int32),
            pltpu.VMEM((EMB_DIM,), jnp.float32),
            pltpu.VMEM((EMB_DIM,), jnp.float32),
            pltpu.SemaphoreType.DMA((2,)),
        ],
    )
    def k(t_hbm, tp_hbm, p_hbm, out_hbm, shared, idx, buf0, buf1, sem):
        c = lax.axis_index("c")
        s = lax.axis_index("s")
        w = s * nc + c
        base = w * rows_per_w

        pltpu.sync_copy(t_hbm.at[pl.ds(s * lr * EMB_DIM, lr * EMB_DIM)],
                        shared.at[pl.ds(s * lr * EMB_DIM, lr * EMB_DIM)])
        pltpu.sync_copy(p_hbm.at[w], idx)
        plsc.subcore_barrier()

        bufs = (buf0, buf1)

        def fire(j, b):
            ix = idx[j][0]

            @pl.when(ix < CACHED_R)
            def _():
                pltpu.async_copy(shared.at[pl.ds(ix * EMB_DIM, EMB_DIM)],
                                 bufs[b], sem.at[b])

            @pl.when(ix >= CACHED_R)
            def _():
                pltpu.async_copy(tp_hbm.at[pl.ds(ix * EMB_DIM, EMB_DIM)],
                                 bufs[b], sem.at[b])

        fire(0, 0)

        def pair(j2, carry):
            for b in range(2):
                j = 2 * j2 + b
                nxt = j + 1

                @pl.when(nxt < rows_per_w)
                def _():
                    fire(nxt, 1 - b)

                pltpu.make_async_copy(t_hbm.at[pl.ds(0, EMB_DIM)],
                                      bufs[b], sem.at[b]).wait()
                pltpu.sync_copy(
                    bufs[b],
                    out_hbm.at[pl.ds((base + j) * EMB_DIM, EMB_DIM)])
            return carry

        lax.fori_loop(0, rows_per_w // 2, pair, 0)

    return k(table1, tpad1, pref)


def kernel(prefix, table):
    info = plsc.get_sparse_core_info()
    nw = info.num_cores * info.num_subcores
    rows_per_w = ROWS // nw
    p3 = prefix.astype(jnp.int32).reshape(nw, rows_per_w, 1)
    pref = jnp.pad(p3, ((0, 0), (0, 0), (0, 15)))
    tpad = jnp.pad(table, ((0, PRE_SEQ_LEN), (0, 0)))
    out = _sc_gather(table.reshape(-1), tpad.reshape(-1), pref)
    return out.reshape(BATCH, PRE_SEQ_LEN, EMB_DIM)


# speedup vs baseline: 2.2812x; 2.2812x over previous
import functools

import jax
import jax.numpy as jnp
from jax import lax
from jax.experimental import pallas as pl
from jax.experimental.pallas import tpu as pltpu
from jax.experimental.pallas import tpu_sc as plsc

PRE_SEQ_LEN = 128
HIDDEN = 768
EMB_DIM = 24 * HIDDEN
BATCH = 32

CH = 2
NCHUNK = PRE_SEQ_LEN // CH


def _sc_gather(table, prefix3):
    info = plsc.get_sparse_core_info()
    nc, ns = info.num_cores, info.num_subcores
    nw = nc * ns
    rows_per_w = (BATCH * PRE_SEQ_LEN) // nw
    mesh = plsc.VectorSubcoreMesh(core_axis_name="c", subcore_axis_name="s")

    @functools.partial(
        pl.kernel,
        out_type=jax.ShapeDtypeStruct((BATCH * PRE_SEQ_LEN, EMB_DIM),
                                      jnp.float32),
        mesh=mesh,
        scratch_types=[
            pltpu.VMEM((NCHUNK, CH), jnp.int32),
            pltpu.VMEM((2, CH, EMB_DIM), jnp.float32),
            pltpu.SemaphoreType.DMA((2,)),
        ],
    )
    def k(table_hbm, pref_hbm, out_hbm, idx_v, buf_v, sem):
        wid = lax.axis_index("s") * nc + lax.axis_index("c")
        base = wid * rows_per_w
        pltpu.sync_copy(pref_hbm.at[wid], idx_v)
        pltpu.async_copy(table_hbm.at[idx_v.at[0]], buf_v.at[0], sem.at[0])

        def pair(j2, carry):
            for b in range(2):
                j = 2 * j2 + b
                nxt = j + 1

                @pl.when(nxt < NCHUNK)
                def _():
                    pltpu.async_copy(table_hbm.at[idx_v.at[nxt]],
                                     buf_v.at[1 - b], sem.at[1 - b])

                pltpu.make_async_copy(table_hbm.at[idx_v.at[j]],
                                      buf_v.at[b], sem.at[b]).wait()
                pltpu.sync_copy(buf_v.at[b],
                                out_hbm.at[pl.ds(base + j * CH, CH)])
            return carry

        lax.fori_loop(0, NCHUNK // 2, pair, 0)

    return k(table, prefix3)


def kernel(prefix, table):
    pref3 = prefix.astype(jnp.int32).reshape(BATCH, NCHUNK, CH)
    out = _sc_gather(table, pref3)
    return out.reshape(BATCH, PRE_SEQ_LEN, EMB_DIM)
